# baseline (speedup 1.0000x reference)
import functools

import jax
import jax.numpy as jnp
from jax import lax
from jax.experimental import pallas as pl
from jax.experimental.pallas import tpu as pltpu
from jax.experimental.pallas import tpu_sc as plsc

N_NODES = 10000
N_EDGES = 320000
D = 128

NC = 2
NS = 16
K = 80
DEGW = 128

ROWS_PER_TILE = N_NODES // NS

DEG_EPT = N_EDGES // NS
DEG_STEPS = DEG_EPT // K

AGG_K = 80
AGG_EPT = 10240
AGG_STEPS = AGG_EPT // AGG_K
AGG_IB = 32
AGG_NB = AGG_STEPS // AGG_IB
PAD_EDGES = NC * NS * AGG_EPT - N_EDGES
AGG_ROWS = N_NODES + 1024

_MESH = plsc.VectorSubcoreMesh(core_axis_name="c", subcore_axis_name="s")


@functools.partial(
    pl.kernel,
    out_type=jax.ShapeDtypeStruct((NC * NS, ROWS_PER_TILE, DEGW), jnp.float32),
    mesh=_MESH,
    scratch_types=[
        pltpu.VMEM((DEG_STEPS, K), jnp.int32),
        pltpu.VMEM((K, DEGW), jnp.float32),
        pltpu.VMEM_SHARED((N_NODES, DEGW), jnp.float32),
    ],
)
def _deg_kernel(idx_hbm, ones_hbm, zeros_hbm, out_hbm, idx_v, ones_v, deg_sh):
    c = lax.axis_index("c")
    s = lax.axis_index("s")
    wid = c * NS + s

    @pl.when(s == 0)
    def _():
        pltpu.sync_copy(zeros_hbm, deg_sh)

    pltpu.sync_copy(idx_hbm.at[wid], idx_v)
    pltpu.sync_copy(ones_hbm, ones_v)
    plsc.subcore_barrier()

    @pl.loop(0, DEG_STEPS)
    def _(j):
        pltpu.sync_copy(ones_v, deg_sh.at[idx_v.at[j]], add=True)

    plsc.subcore_barrier()
    pltpu.sync_copy(deg_sh.at[pl.ds(s * ROWS_PER_TILE, ROWS_PER_TILE)],
                    out_hbm.at[wid])


@functools.partial(
    pl.kernel,
    out_type=jax.ShapeDtypeStruct((NC * NS, ROWS_PER_TILE, D), jnp.float32),
    mesh=_MESH,
    scratch_types=[
        pltpu.VMEM((AGG_IB, AGG_K), jnp.int32),
        pltpu.VMEM((AGG_IB, AGG_K), jnp.int32),
        pltpu.VMEM((AGG_K, D), jnp.float32),
        pltpu.VMEM((AGG_K, D), jnp.float32),
        pltpu.VMEM_SHARED((AGG_ROWS, D), jnp.float32),
        pltpu.SemaphoreType.DMA,
        pltpu.SemaphoreType.DMA,
    ],
)
def _agg_kernel(h_hbm, src_hbm, dst_hbm, zeros_hbm, out_hbm,
                sidx_v, didx_v, rows0_v, rows1_v, agg_sh, sem0, sem1):
    c = lax.axis_index("c")
    s = lax.axis_index("s")
    wid = c * NS + s

    @pl.when(s == 0)
    def _():
        pltpu.sync_copy(zeros_hbm, agg_sh)

    plsc.subcore_barrier()

    @pl.loop(0, AGG_NB)
    def _(blk):
        pltpu.sync_copy(src_hbm.at[wid].at[blk], sidx_v)
        pltpu.sync_copy(dst_hbm.at[wid].at[blk], didx_v)
        pltpu.async_copy(h_hbm.at[sidx_v.at[0]], rows0_v, sem0)

        @pl.loop(0, AGG_IB // 2)
        def _(j2):
            j0 = 2 * j2
            j1 = j0 + 1
            pltpu.async_copy(h_hbm.at[sidx_v.at[j1]], rows1_v, sem1)
            pltpu.make_async_copy(h_hbm.at[sidx_v.at[j0]], rows0_v,
                                  sem0).wait()
            pltpu.sync_copy(rows0_v, agg_sh.at[didx_v.at[j0]], add=True)

            @pl.when(j0 + 2 < AGG_IB)
            def _():
                pltpu.async_copy(h_hbm.at[sidx_v.at[j0 + 2]], rows0_v, sem0)

            pltpu.make_async_copy(h_hbm.at[sidx_v.at[j1]], rows1_v,
                                  sem1).wait()
            pltpu.sync_copy(rows1_v, agg_sh.at[didx_v.at[j1]], add=True)

    plsc.subcore_barrier()
    pltpu.sync_copy(agg_sh.at[pl.ds(s * ROWS_PER_TILE, ROWS_PER_TILE)],
                    out_hbm.at[wid])


_ROWS_BLK = 1000


def _mm_body(feat_ref, w_ref, h_ref):
    h_ref[...] = jnp.dot(feat_ref[...], w_ref[...],
                         preferred_element_type=jnp.float32)


def _mm(features, W):
    return pl.pallas_call(
        _mm_body,
        grid=(N_NODES // _ROWS_BLK,),
        in_specs=[
            pl.BlockSpec((_ROWS_BLK, D), lambda i: (i, 0)),
            pl.BlockSpec((D, D), lambda i: (0, 0)),
        ],
        out_specs=pl.BlockSpec((_ROWS_BLK, D), lambda i: (i, 0)),
        out_shape=jax.ShapeDtypeStruct((N_NODES, D), jnp.float32),
    )(features, W)


def _scale_body(mm_ref, deg_ref, h_ref):
    deg = deg_ref[0]
    norm = jnp.where(deg > 0.0, lax.rsqrt(jnp.maximum(deg, 1.0)), 0.0)
    h_ref[...] = mm_ref[...] * norm[:, 0:1]


def _scale(mm, degs):
    return pl.pallas_call(
        _scale_body,
        grid=(N_NODES // _ROWS_BLK,),
        in_specs=[
            pl.BlockSpec((_ROWS_BLK, D), lambda i: (i, 0)),
            pl.BlockSpec((1, _ROWS_BLK, DEGW), lambda i: (0, i, 0)),
        ],
        out_specs=pl.BlockSpec((_ROWS_BLK, D), lambda i: (i, 0)),
        out_shape=jax.ShapeDtypeStruct((N_NODES, D), jnp.float32),
    )(mm, degs)


def _final_body(agg_ref, deg_ref, b_ref, out_ref):
    a = agg_ref[0] + agg_ref[1]
    deg = deg_ref[0]
    norm = jnp.where(deg > 0.0, lax.rsqrt(jnp.maximum(deg, 1.0)), 0.0)
    y = a * norm[:, 0:1] + b_ref[...]
    out_ref[...] = jnp.where(y > 0.0, y, jnp.exp(jnp.minimum(y, 0.0)) - 1.0)


def _final(aggs, degs, b):
    return pl.pallas_call(
        _final_body,
        grid=(N_NODES // _ROWS_BLK,),
        in_specs=[
            pl.BlockSpec((NC, _ROWS_BLK, D), lambda i: (0, i, 0)),
            pl.BlockSpec((1, _ROWS_BLK, DEGW), lambda i: (1, i, 0)),
            pl.BlockSpec((1, D), lambda i: (0, 0)),
        ],
        out_specs=pl.BlockSpec((_ROWS_BLK, D), lambda i: (i, 0)),
        out_shape=jax.ShapeDtypeStruct((N_NODES, D), jnp.float32),
    )(aggs, degs, b)


@jax.jit
def kernel(features, edge_index, W, b):
    src = edge_index[0].astype(jnp.int32)
    dst = edge_index[1].astype(jnp.int32)

    idx_all = jnp.stack([src, dst]).reshape(NC * NS, DEG_STEPS, K)
    ones16 = jnp.ones((K, DEGW), jnp.float32)
    zeros16 = jnp.zeros((N_NODES, DEGW), jnp.float32)
    mm = _mm(features, W)
    degs = _deg_kernel(idx_all, ones16, zeros16)
    degs = degs.reshape(NC, N_NODES, DEGW)

    h = _scale(mm, degs)

    src_t = jnp.concatenate(
        [src, jnp.arange(PAD_EDGES, dtype=jnp.int32) % N_NODES]).reshape(
            NC * NS, AGG_NB, AGG_IB, AGG_K)
    dst_t = jnp.concatenate(
        [dst, N_NODES + jnp.arange(PAD_EDGES, dtype=jnp.int32) % 1024]).reshape(
            NC * NS, AGG_NB, AGG_IB, AGG_K)
    zeros = jnp.zeros((AGG_ROWS, D), jnp.float32)
    aggs = _agg_kernel(h, src_t, dst_t, zeros)
    aggs = aggs.reshape(NC, N_NODES, D)

    return _final(aggs, degs, b.reshape(1, D))


# speedup vs baseline: 1.0262x; 1.0262x over previous
import functools

import jax
import jax.numpy as jnp
from jax import lax
from jax.experimental import pallas as pl
from jax.experimental.pallas import tpu as pltpu
from jax.experimental.pallas import tpu_sc as plsc

N_NODES = 10000
N_EDGES = 320000
D = 128

NC = 2
NS = 16
K = 80
DEGW = 128

ROWS_PER_TILE = N_NODES // NS

DEG_EPT = N_EDGES // NS
DEG_STEPS = DEG_EPT // K

AGG_K = 128
AGG_EPT = 10240
AGG_STEPS = AGG_EPT // AGG_K
AGG_IB = 16
AGG_NB = AGG_STEPS // AGG_IB
PAD_EDGES = NC * NS * AGG_EPT - N_EDGES
AGG_ROWS = N_NODES + 1024

_MESH = plsc.VectorSubcoreMesh(core_axis_name="c", subcore_axis_name="s")


@functools.partial(
    pl.kernel,
    out_type=jax.ShapeDtypeStruct((NC * NS, ROWS_PER_TILE, DEGW), jnp.float32),
    mesh=_MESH,
    scratch_types=[
        pltpu.VMEM((DEG_STEPS, K), jnp.int32),
        pltpu.VMEM((K, DEGW), jnp.float32),
        pltpu.VMEM_SHARED((N_NODES, DEGW), jnp.float32),
    ],
)
def _deg_kernel(idx_hbm, ones_hbm, zeros_hbm, out_hbm, idx_v, ones_v, deg_sh):
    c = lax.axis_index("c")
    s = lax.axis_index("s")
    wid = c * NS + s

    @pl.when(s == 0)
    def _():
        pltpu.sync_copy(zeros_hbm, deg_sh)

    pltpu.sync_copy(idx_hbm.at[wid], idx_v)
    pltpu.sync_copy(ones_hbm, ones_v)
    plsc.subcore_barrier()

    @pl.loop(0, DEG_STEPS)
    def _(j):
        pltpu.sync_copy(ones_v, deg_sh.at[idx_v.at[j]], add=True)

    plsc.subcore_barrier()
    pltpu.sync_copy(deg_sh.at[pl.ds(s * ROWS_PER_TILE, ROWS_PER_TILE)],
                    out_hbm.at[wid])


@functools.partial(
    pl.kernel,
    out_type=jax.ShapeDtypeStruct((NC * NS, ROWS_PER_TILE, D), jnp.float32),
    mesh=_MESH,
    scratch_types=[
        pltpu.VMEM((AGG_IB, AGG_K), jnp.int32),
        pltpu.VMEM((AGG_IB, AGG_K), jnp.int32),
        pltpu.VMEM((AGG_K, D), jnp.float32),
        pltpu.VMEM((AGG_K, D), jnp.float32),
        pltpu.VMEM_SHARED((AGG_ROWS, D), jnp.float32),
        pltpu.SemaphoreType.DMA,
        pltpu.SemaphoreType.DMA,
    ],
)
def _agg_kernel(h_hbm, src_hbm, dst_hbm, zeros_hbm, out_hbm,
                sidx_v, didx_v, rows0_v, rows1_v, agg_sh, sem0, sem1):
    c = lax.axis_index("c")
    s = lax.axis_index("s")
    wid = c * NS + s

    @pl.when(s == 0)
    def _():
        pltpu.sync_copy(zeros_hbm, agg_sh)

    plsc.subcore_barrier()

    @pl.loop(0, AGG_NB)
    def _(blk):
        pltpu.sync_copy(src_hbm.at[wid].at[blk], sidx_v)
        pltpu.sync_copy(dst_hbm.at[wid].at[blk], didx_v)
        pltpu.async_copy(h_hbm.at[sidx_v.at[0]], rows0_v, sem0)

        @pl.loop(0, AGG_IB // 2)
        def _(j2):
            j0 = 2 * j2
            j1 = j0 + 1
            pltpu.async_copy(h_hbm.at[sidx_v.at[j1]], rows1_v, sem1)
            pltpu.make_async_copy(h_hbm.at[sidx_v.at[j0]], rows0_v,
                                  sem0).wait()
            pltpu.sync_copy(rows0_v, agg_sh.at[didx_v.at[j0]], add=True)

            @pl.when(j0 + 2 < AGG_IB)
            def _():
                pltpu.async_copy(h_hbm.at[sidx_v.at[j0 + 2]], rows0_v, sem0)

            pltpu.make_async_copy(h_hbm.at[sidx_v.at[j1]], rows1_v,
                                  sem1).wait()
            pltpu.sync_copy(rows1_v, agg_sh.at[didx_v.at[j1]], add=True)

    plsc.subcore_barrier()
    pltpu.sync_copy(agg_sh.at[pl.ds(s * ROWS_PER_TILE, ROWS_PER_TILE)],
                    out_hbm.at[wid])


_ROWS_BLK = 1000


def _prep_body(feat_ref, deg_ref, w_ref, h_ref):
    deg = deg_ref[0]
    norm = jnp.where(deg > 0.0, lax.rsqrt(jnp.maximum(deg, 1.0)), 0.0)
    x = feat_ref[...] * norm[:, 0:1]
    h_ref[...] = jnp.dot(x, w_ref[...], preferred_element_type=jnp.float32)


def _prep(features, degs, W):
    return pl.pallas_call(
        _prep_body,
        grid=(N_NODES // _ROWS_BLK,),
        in_specs=[
            pl.BlockSpec((_ROWS_BLK, D), lambda i: (i, 0)),
            pl.BlockSpec((1, _ROWS_BLK, DEGW), lambda i: (0, i, 0)),
            pl.BlockSpec((D, D), lambda i: (0, 0)),
        ],
        out_specs=pl.BlockSpec((_ROWS_BLK, D), lambda i: (i, 0)),
        out_shape=jax.ShapeDtypeStruct((N_NODES, D), jnp.float32),
    )(features, degs, W)


def _final_body(agg_ref, deg_ref, b_ref, out_ref):
    a = agg_ref[0] + agg_ref[1]
    deg = deg_ref[0]
    norm = jnp.where(deg > 0.0, lax.rsqrt(jnp.maximum(deg, 1.0)), 0.0)
    y = a * norm[:, 0:1] + b_ref[...]
    out_ref[...] = jnp.where(y > 0.0, y, jnp.exp(jnp.minimum(y, 0.0)) - 1.0)


def _final(aggs, degs, b):
    return pl.pallas_call(
        _final_body,
        grid=(N_NODES // _ROWS_BLK,),
        in_specs=[
            pl.BlockSpec((NC, _ROWS_BLK, D), lambda i: (0, i, 0)),
            pl.BlockSpec((1, _ROWS_BLK, DEGW), lambda i: (1, i, 0)),
            pl.BlockSpec((1, D), lambda i: (0, 0)),
        ],
        out_specs=pl.BlockSpec((_ROWS_BLK, D), lambda i: (i, 0)),
        out_shape=jax.ShapeDtypeStruct((N_NODES, D), jnp.float32),
    )(aggs, degs, b)


@jax.jit
def kernel(features, edge_index, W, b):
    src = edge_index[0].astype(jnp.int32)
    dst = edge_index[1].astype(jnp.int32)

    idx_all = jnp.stack([src, dst]).reshape(NC * NS, DEG_STEPS, K)
    ones16 = jnp.ones((K, DEGW), jnp.float32)
    zeros16 = jnp.zeros((N_NODES, DEGW), jnp.float32)
    degs = _deg_kernel(idx_all, ones16, zeros16)
    degs = degs.reshape(NC, N_NODES, DEGW)

    h = _prep(features, degs, W)

    src_t = jnp.concatenate(
        [src, jnp.arange(PAD_EDGES, dtype=jnp.int32) % N_NODES]).reshape(
            NC * NS, AGG_NB, AGG_IB, AGG_K)
    dst_t = jnp.concatenate(
        [dst, N_NODES + jnp.arange(PAD_EDGES, dtype=jnp.int32) % 1024]).reshape(
            NC * NS, AGG_NB, AGG_IB, AGG_K)
    zeros = jnp.zeros((AGG_ROWS, D), jnp.float32)
    aggs = _agg_kernel(h, src_t, dst_t, zeros)
    aggs = aggs.reshape(NC, N_NODES, D)

    return _final(aggs, degs, b.reshape(1, D))
